# baseline (speedup 1.0000x reference)
import jax
import jax.numpy as jnp
from jax.experimental import pallas as pl
from jax.experimental.pallas import tpu as pltpu

_NBUF = 3


def _pool_block(x, w, bias):
    logits = jnp.sum(x * w, axis=2, keepdims=True) + bias
    logits = logits - jnp.max(logits, axis=1, keepdims=True)
    e = jnp.exp(logits)
    p = e / jnp.sum(e, axis=1, keepdims=True)
    return jnp.sum(p * x, axis=1, keepdims=True)


def _seqpool_body(x_hbm, w_ref, b_ref, o_ref, buf, sem):
    step = pl.program_id(0)
    nsteps = pl.num_programs(0)
    tb = buf.shape[1]

    def fetch(i, slot):
        pltpu.make_async_copy(
            x_hbm.at[pl.ds(i * tb, tb)], buf.at[slot], sem.at[slot]
        ).start()

    @pl.when(step == 0)
    def _prologue():
        for j in range(_NBUF):
            @pl.when(j < nsteps)
            def _():
                fetch(j, j)

    slot = jax.lax.rem(step, _NBUF)
    pltpu.make_async_copy(
        x_hbm.at[pl.ds(step * tb, tb)], buf.at[slot], sem.at[slot]
    ).wait()
    o_ref[...] = _pool_block(buf[slot], w_ref[...], b_ref[0]).astype(o_ref.dtype)

    nxt = step + _NBUF
    @pl.when(nxt < nsteps)
    def _refill():
        fetch(nxt, slot)


def _pick_batch_tile(B):
    for tb in range(min(512, B), 0, -8):
        if B % tb == 0:
            return tb
    return 1


def kernel(x, w, b):
    B, N, D = x.shape
    tb = _pick_batch_tile(B)
    grid = (B // tb,)
    out = pl.pallas_call(
        _seqpool_body,
        out_shape=jax.ShapeDtypeStruct((B, 1, D), x.dtype),
        grid=grid,
        in_specs=[
            pl.BlockSpec(memory_space=pltpu.MemorySpace.HBM),
            pl.BlockSpec(memory_space=pltpu.MemorySpace.VMEM),
            pl.BlockSpec(memory_space=pltpu.MemorySpace.SMEM),
        ],
        out_specs=pl.BlockSpec((tb, 1, D), lambda i: (i, 0, 0)),
        scratch_shapes=[
            pltpu.VMEM((_NBUF, tb, N, D), x.dtype),
            pltpu.SemaphoreType.DMA((_NBUF,)),
        ],
        compiler_params=pltpu.CompilerParams(
            dimension_semantics=("arbitrary",),
            vmem_limit_bytes=64 * 1024 * 1024,
        ),
    )(x, w, b)
    return out


# speedup vs baseline: 1.0090x; 1.0090x over previous
import jax
import jax.numpy as jnp
from jax.experimental import pallas as pl
from jax.experimental.pallas import tpu as pltpu


def _seqpool_body(x_ref, w_ref, b_ref, o_ref):
    x = x_ref[...]
    logits = jnp.sum(x * w_ref[...], axis=2, keepdims=True) + b_ref[0]
    logits = logits - jnp.max(logits, axis=1, keepdims=True)
    e = jnp.exp(logits)
    p = e / jnp.sum(e, axis=1, keepdims=True)
    o_ref[...] = jnp.sum(p * x, axis=1, keepdims=True).astype(o_ref.dtype)


def _pick_batch_tile(B):
    for tb in range(min(512, B), 0, -8):
        if B % tb == 0:
            return tb
    return 1


def kernel(x, w, b):
    B, N, D = x.shape
    tb = _pick_batch_tile(B)
    grid = (B // tb,)
    out = pl.pallas_call(
        _seqpool_body,
        out_shape=jax.ShapeDtypeStruct((B, 1, D), x.dtype),
        grid=grid,
        in_specs=[
            pl.BlockSpec((tb, N, D), lambda i: (i, 0, 0)),
            pl.BlockSpec(memory_space=pltpu.MemorySpace.VMEM),
            pl.BlockSpec(memory_space=pltpu.MemorySpace.SMEM),
        ],
        out_specs=pl.BlockSpec((tb, 1, D), lambda i: (i, 0, 0)),
        compiler_params=pltpu.CompilerParams(
            dimension_semantics=("parallel",),
            vmem_limit_bytes=64 * 1024 * 1024,
        ),
    )(x, w, b)
    return out
